# Initial kernel scaffold; baseline (speedup 1.0000x reference)
import jax
import jax.numpy as jnp
from jax.experimental import pallas as pl


def kernel(x, edge_index, W1a, b1a, W1b, b1b, W2a, b2a, W2b, b2b):
    raise NotImplementedError("write your pallas kernel here")


# speedup vs baseline: 16.5645x; 16.5645x over previous
import functools

import jax
import jax.numpy as jnp
from jax import lax
from jax.experimental import pallas as pl
from jax.experimental.pallas import tpu as pltpu
from jax.experimental.pallas import tpu_sc as plsc

N = 10000
NPAD = 10240
E = 320000
EPAD = 327680
NC, NS = 2, 16
CHUNK = 128

_mesh = plsc.VectorSubcoreMesh(core_axis_name="c", subcore_axis_name="s")
_sc_params = pltpu.CompilerParams(use_tc_tiling_on_sc=False)


@functools.partial(
    pl.kernel,
    out_type=jax.ShapeDtypeStruct((2 * NPAD, 8), jnp.float32),
    mesh=_mesh,
    compiler_params=_sc_params,
    scratch_types=[
        pltpu.VMEM((80, CHUNK), jnp.int32),
        pltpu.VMEM((CHUNK, 8), jnp.float32),
        pltpu.VMEM((640, 8), jnp.float32),
        pltpu.VMEM_SHARED((NPAD, 8), jnp.float32),
        pltpu.SemaphoreType.DMA,
    ],
)
def _deg_k(dst_hbm, ones_hbm, half_hbm, out_hbm, dst_v, ones_v, stage_v,
           acc_s, sem):
    c = lax.axis_index("c")
    s = lax.axis_index("s")
    tid = c * NS + s
    pltpu.sync_copy(dst_hbm.at[tid], dst_v)
    pltpu.sync_copy(ones_hbm, ones_v)
    pltpu.sync_copy(half_hbm.at[pl.ds(s * 640, 640)], stage_v)
    pltpu.sync_copy(stage_v, acc_s.at[pl.ds(s * 640, 640)])
    plsc.subcore_barrier()

    def body(j, carry):
        pltpu.sync_copy(ones_v, acc_s.at[dst_v.at[j]], add=True)
        return carry

    lax.fori_loop(0, 80, body, 0)
    plsc.subcore_barrier()
    pltpu.sync_copy(acc_s.at[pl.ds(s * 640, 640)], stage_v)
    pltpu.sync_copy(stage_v, out_hbm.at[pl.ds(c * NPAD + s * 640, 640)])


def _make_prop(F):
    rows_per_tile = NPAD // NS

    @functools.partial(
        pl.kernel,
        out_type=jax.ShapeDtypeStruct((2 * NPAD, F), jnp.float32),
        mesh=_mesh,
        compiler_params=_sc_params,
        scratch_types=[
            pltpu.VMEM((160, CHUNK), jnp.int32),
            pltpu.VMEM((160, CHUNK), jnp.int32),
            pltpu.VMEM((CHUNK, F), jnp.float32),
            pltpu.VMEM((rows_per_tile, F), jnp.float32),
            pltpu.VMEM_SHARED((NPAD, F), jnp.float32),
            pltpu.SemaphoreType.DMA,
        ],
    )
    def prop(u_hbm, src_hbm, dst_hbm, out_hbm, src_v, dst_v, rows_v, stage_v,
             acc_s, sem):
        c = lax.axis_index("c")
        s = lax.axis_index("s")
        tid = c * NS + s
        base = s * rows_per_tile
        pltpu.sync_copy(src_hbm.at[tid], src_v)
        pltpu.sync_copy(dst_hbm.at[tid], dst_v)
        pltpu.sync_copy(u_hbm.at[pl.ds(c * NPAD + base, rows_per_tile)],
                        stage_v)
        pltpu.sync_copy(stage_v, acc_s.at[pl.ds(base, rows_per_tile)])
        plsc.subcore_barrier()

        def body(j, carry):
            pltpu.async_copy(u_hbm.at[src_v.at[j]], rows_v, sem).wait()
            pltpu.sync_copy(rows_v, acc_s.at[dst_v.at[j]], add=True)
            return carry

        lax.fori_loop(0, 160, body, 0)
        plsc.subcore_barrier()
        pltpu.sync_copy(acc_s.at[pl.ds(base, rows_per_tile)], stage_v)
        pltpu.sync_copy(stage_v, out_hbm.at[pl.ds(c * NPAD + base,
                                                  rows_per_tile)])

    return prop


_prop64 = _make_prop(64)
_prop32 = _make_prop(32)


def _scale_body(degT_ref, x_ref, u_ref, dis_ref):
    deg = degT_ref[:, 0:1] + degT_ref[:, 1:2]
    dis = lax.rsqrt(jnp.maximum(deg, 1e-12))
    dis_ref[...] = jnp.broadcast_to(dis, (NPAD, 8))
    u_ref[0] = x_ref[:, :64] * dis
    u_ref[1] = x_ref[:, 64:] * dis


_scale_k = pl.pallas_call(
    _scale_body,
    out_shape=(
        jax.ShapeDtypeStruct((2, NPAD, 64), jnp.float32),
        jax.ShapeDtypeStruct((NPAD, 8), jnp.float32),
    ),
)


_ROWS = 256


def _mm_body(w_ref, dis_ref, W1a_ref, b1a_ref, W1b_ref, W2a_ref, b2a_ref,
             W2b_ref, o_ref):
    dis = dis_ref[:, 0:1]
    z = jnp.concatenate([w_ref[0], w_ref[1]], axis=1) * dis
    h1 = jnp.maximum(
        jnp.dot(z, W1a_ref[...], preferred_element_type=jnp.float32)
        + b1a_ref[...], 0.0)
    g1 = jnp.dot(h1, W1b_ref[...], preferred_element_type=jnp.float32)
    o_ref[0] = g1 * dis
    h2 = jnp.maximum(
        jnp.dot(z, W2a_ref[...], preferred_element_type=jnp.float32)
        + b2a_ref[...], 0.0)
    g2 = jnp.dot(h2, W2b_ref[...], preferred_element_type=jnp.float32)
    o_ref[1] = g2 * dis


_mm_k = pl.pallas_call(
    _mm_body,
    grid=(NPAD // _ROWS,),
    in_specs=[
        pl.BlockSpec((2, _ROWS, 64), lambda i: (0, i, 0)),
        pl.BlockSpec((_ROWS, 8), lambda i: (i, 0)),
        pl.BlockSpec((128, 256), lambda i: (0, 0)),
        pl.BlockSpec((1, 256), lambda i: (0, 0)),
        pl.BlockSpec((256, 32), lambda i: (0, 0)),
        pl.BlockSpec((128, 256), lambda i: (0, 0)),
        pl.BlockSpec((1, 256), lambda i: (0, 0)),
        pl.BlockSpec((256, 32), lambda i: (0, 0)),
    ],
    out_specs=pl.BlockSpec((2, _ROWS, 32), lambda i: (0, i, 0)),
    out_shape=jax.ShapeDtypeStruct((2, NPAD, 32), jnp.float32),
)


def _fin_body(w2_ref, dis_ref, b1b_ref, b2b_ref, o1_ref, o2_ref):
    dis = dis_ref[:, 0:1]
    o1_ref[...] = w2_ref[0] * dis + b1b_ref[...]
    o2_ref[...] = w2_ref[1] * dis + b2b_ref[...]


_fin_k = pl.pallas_call(
    _fin_body,
    out_shape=(
        jax.ShapeDtypeStruct((NPAD, 32), jnp.float32),
        jax.ShapeDtypeStruct((NPAD, 32), jnp.float32),
    ),
)


def kernel(x, edge_index, W1a, b1a, W1b, b1b, W2a, b2a, W2b, b2b):
    src = edge_index[0]
    dst = edge_index[1]
    pad = jnp.full((EPAD - E,), N, jnp.int32)
    srcp = jnp.concatenate([src, pad])
    dstp = jnp.concatenate([dst, pad])
    dstA = dstp.reshape(32, 80, CHUNK)
    src16 = srcp.reshape(16, 160, CHUNK)
    dst16 = dstp.reshape(16, 160, CHUNK)
    srcT = jnp.concatenate([src16, src16 + NPAD], axis=0)
    dstT = jnp.concatenate([dst16, dst16], axis=0)
    x_pad = jnp.pad(x, ((0, NPAD - N), (0, 0)))

    degpair = _deg_k(dstA, jnp.full((CHUNK, 8), 1.0, jnp.float32),
                     jnp.full((NPAD, 8), 0.5, jnp.float32))
    degT = degpair.reshape(2, NPAD, 8)[:, :, 0].T
    u, dis8 = _scale_k(degT, x_pad)
    w = _prop64(u.reshape(2 * NPAD, 64), srcT, dstT)
    u2 = _mm_k(w.reshape(2, NPAD, 64), dis8, W1a, b1a.reshape(1, -1), W1b,
               W2a, b2a.reshape(1, -1), W2b)
    w2 = _prop32(u2.reshape(2 * NPAD, 32), srcT, dstT)
    o1, o2 = _fin_k(w2.reshape(2, NPAD, 32), dis8, b1b.reshape(1, -1),
                    b2b.reshape(1, -1))
    return (o1[:N], o2[:N])


# speedup vs baseline: 18.1273x; 1.0943x over previous
import functools

import jax
import jax.numpy as jnp
from jax import lax
from jax.experimental import pallas as pl
from jax.experimental.pallas import tpu as pltpu
from jax.experimental.pallas import tpu_sc as plsc

N = 10000
NPAD = 10240
E = 320000
EPAD = 327680
NC, NS = 2, 16
CHUNK = 128

_mesh = plsc.VectorSubcoreMesh(core_axis_name="c", subcore_axis_name="s")
_sc_params = pltpu.CompilerParams(use_tc_tiling_on_sc=False)


@functools.partial(
    pl.kernel,
    out_type=jax.ShapeDtypeStruct((2 * NPAD, 8), jnp.float32),
    mesh=_mesh,
    compiler_params=_sc_params,
    scratch_types=[
        pltpu.VMEM((80, CHUNK), jnp.int32),
        pltpu.VMEM((CHUNK, 8), jnp.float32),
        pltpu.VMEM((640, 8), jnp.float32),
        pltpu.VMEM_SHARED((NPAD, 8), jnp.float32),
        pltpu.SemaphoreType.DMA,
    ],
)
def _deg_k(dst_hbm, ones_hbm, half_hbm, out_hbm, dst_v, ones_v, stage_v,
           acc_s, sem):
    c = lax.axis_index("c")
    s = lax.axis_index("s")
    tid = c * NS + s
    pltpu.sync_copy(dst_hbm.at[tid], dst_v)
    pltpu.sync_copy(ones_hbm, ones_v)
    pltpu.sync_copy(half_hbm.at[pl.ds(s * 640, 640)], stage_v)
    pltpu.sync_copy(stage_v, acc_s.at[pl.ds(s * 640, 640)])
    plsc.subcore_barrier()

    def body(j, carry):
        pltpu.sync_copy(ones_v, acc_s.at[dst_v.at[j]], add=True)
        return carry

    lax.fori_loop(0, 80, body, 0)
    plsc.subcore_barrier()
    pltpu.sync_copy(acc_s.at[pl.ds(s * 640, 640)], stage_v)
    pltpu.sync_copy(stage_v, out_hbm.at[pl.ds(c * NPAD + s * 640, 640)])


def _make_prop(F):
    rows_per_tile = NPAD // NS

    @functools.partial(
        pl.kernel,
        out_type=jax.ShapeDtypeStruct((2 * NPAD, F), jnp.float32),
        mesh=_mesh,
        compiler_params=_sc_params,
        scratch_types=[
            pltpu.VMEM((160, CHUNK), jnp.int32),
            pltpu.VMEM((160, CHUNK), jnp.int32),
            pltpu.VMEM((CHUNK, F), jnp.float32),
            pltpu.VMEM((CHUNK, F), jnp.float32),
            pltpu.VMEM_SHARED((NPAD, F), jnp.float32),
            pltpu.SemaphoreType.DMA,
            pltpu.SemaphoreType.DMA,
        ],
    )
    def prop(u_hbm, src_hbm, dst_hbm, out_hbm, src_v, dst_v, rows0_v, rows1_v,
             acc_s, sem0, sem1):
        c = lax.axis_index("c")
        s = lax.axis_index("s")
        tid = c * NS + s
        base = s * rows_per_tile
        pltpu.sync_copy(src_hbm.at[tid], src_v)
        pltpu.sync_copy(dst_hbm.at[tid], dst_v)
        for k in range(rows_per_tile // CHUNK):
            pltpu.sync_copy(
                u_hbm.at[pl.ds(c * NPAD + base + k * CHUNK, CHUNK)], rows0_v)
            pltpu.sync_copy(rows0_v, acc_s.at[pl.ds(base + k * CHUNK, CHUNK)])
        plsc.subcore_barrier()

        pltpu.async_copy(u_hbm.at[src_v.at[0]], rows0_v, sem0)

        def body(g, carry):
            j0 = 2 * g
            j1 = 2 * g + 1
            pltpu.make_async_copy(u_hbm.at[src_v.at[j0]], rows0_v,
                                  sem0).wait()
            pltpu.async_copy(u_hbm.at[src_v.at[j1]], rows1_v, sem1)
            pltpu.sync_copy(rows0_v, acc_s.at[dst_v.at[j0]], add=True)
            pltpu.make_async_copy(u_hbm.at[src_v.at[j1]], rows1_v,
                                  sem1).wait()

            @pl.when(j1 + 1 < 160)
            def _():
                pltpu.async_copy(u_hbm.at[src_v.at[j1 + 1]], rows0_v, sem0)

            pltpu.sync_copy(rows1_v, acc_s.at[dst_v.at[j1]], add=True)
            return carry

        lax.fori_loop(0, 80, body, 0)
        plsc.subcore_barrier()
        for k in range(rows_per_tile // CHUNK):
            pltpu.sync_copy(acc_s.at[pl.ds(base + k * CHUNK, CHUNK)], rows0_v)
            pltpu.sync_copy(
                rows0_v, out_hbm.at[pl.ds(c * NPAD + base + k * CHUNK, CHUNK)])

    return prop


_prop64 = _make_prop(64)
_prop32 = _make_prop(32)


def _scale_body(degT_ref, x_ref, u_ref, dis_ref):
    deg = degT_ref[:, 0:1] + degT_ref[:, 1:2]
    dis = lax.rsqrt(jnp.maximum(deg, 1e-12))
    dis_ref[...] = jnp.broadcast_to(dis, (NPAD, 8))
    u_ref[0] = x_ref[:, :64] * dis
    u_ref[1] = x_ref[:, 64:] * dis


_scale_k = pl.pallas_call(
    _scale_body,
    out_shape=(
        jax.ShapeDtypeStruct((2, NPAD, 64), jnp.float32),
        jax.ShapeDtypeStruct((NPAD, 8), jnp.float32),
    ),
)


_ROWS = 256


def _mm_body(w_ref, dis_ref, W1a_ref, b1a_ref, W1b_ref, W2a_ref, b2a_ref,
             W2b_ref, o_ref):
    dis = dis_ref[:, 0:1]
    z = jnp.concatenate([w_ref[0], w_ref[1]], axis=1) * dis
    h1 = jnp.maximum(
        jnp.dot(z, W1a_ref[...], preferred_element_type=jnp.float32)
        + b1a_ref[...], 0.0)
    g1 = jnp.dot(h1, W1b_ref[...], preferred_element_type=jnp.float32)
    o_ref[0] = g1 * dis
    h2 = jnp.maximum(
        jnp.dot(z, W2a_ref[...], preferred_element_type=jnp.float32)
        + b2a_ref[...], 0.0)
    g2 = jnp.dot(h2, W2b_ref[...], preferred_element_type=jnp.float32)
    o_ref[1] = g2 * dis


_mm_k = pl.pallas_call(
    _mm_body,
    grid=(NPAD // _ROWS,),
    in_specs=[
        pl.BlockSpec((2, _ROWS, 64), lambda i: (0, i, 0)),
        pl.BlockSpec((_ROWS, 8), lambda i: (i, 0)),
        pl.BlockSpec((128, 256), lambda i: (0, 0)),
        pl.BlockSpec((1, 256), lambda i: (0, 0)),
        pl.BlockSpec((256, 32), lambda i: (0, 0)),
        pl.BlockSpec((128, 256), lambda i: (0, 0)),
        pl.BlockSpec((1, 256), lambda i: (0, 0)),
        pl.BlockSpec((256, 32), lambda i: (0, 0)),
    ],
    out_specs=pl.BlockSpec((2, _ROWS, 32), lambda i: (0, i, 0)),
    out_shape=jax.ShapeDtypeStruct((2, NPAD, 32), jnp.float32),
)


def _fin_body(w2_ref, dis_ref, b1b_ref, b2b_ref, o1_ref, o2_ref):
    dis = dis_ref[:, 0:1]
    o1_ref[...] = w2_ref[0] * dis + b1b_ref[...]
    o2_ref[...] = w2_ref[1] * dis + b2b_ref[...]


_fin_k = pl.pallas_call(
    _fin_body,
    out_shape=(
        jax.ShapeDtypeStruct((NPAD, 32), jnp.float32),
        jax.ShapeDtypeStruct((NPAD, 32), jnp.float32),
    ),
)


def kernel(x, edge_index, W1a, b1a, W1b, b1b, W2a, b2a, W2b, b2b):
    src = edge_index[0]
    dst = edge_index[1]
    pad = jnp.full((EPAD - E,), N, jnp.int32)
    srcp = jnp.concatenate([src, pad])
    dstp = jnp.concatenate([dst, pad])
    dstA = dstp.reshape(32, 80, CHUNK)
    src16 = srcp.reshape(16, 160, CHUNK)
    dst16 = dstp.reshape(16, 160, CHUNK)
    srcT = jnp.concatenate([src16, src16 + NPAD], axis=0)
    dstT = jnp.concatenate([dst16, dst16], axis=0)
    x_pad = jnp.pad(x, ((0, NPAD - N), (0, 0)))

    degpair = _deg_k(dstA, jnp.full((CHUNK, 8), 1.0, jnp.float32),
                     jnp.full((NPAD, 8), 0.5, jnp.float32))
    degT = degpair.reshape(2, NPAD, 8)[:, :, 0].T
    u, dis8 = _scale_k(degT, x_pad)
    w = _prop64(u.reshape(2 * NPAD, 64), srcT, dstT)
    u2 = _mm_k(w.reshape(2, NPAD, 64), dis8, W1a, b1a.reshape(1, -1), W1b,
               W2a, b2a.reshape(1, -1), W2b)
    w2 = _prop32(u2.reshape(2 * NPAD, 32), srcT, dstT)
    o1, o2 = _fin_k(w2.reshape(2, NPAD, 32), dis8, b1b.reshape(1, -1),
                    b2b.reshape(1, -1))
    return (o1[:N], o2[:N])


# speedup vs baseline: 20.7542x; 1.1449x over previous
import functools

import jax
import jax.numpy as jnp
from jax import lax
from jax.experimental import pallas as pl
from jax.experimental.pallas import tpu as pltpu
from jax.experimental.pallas import tpu_sc as plsc

N = 10000
NPAD = 10240
E = 320000
EPAD = 327680
NC, NS = 2, 16
CHUNK = 128

_mesh = plsc.VectorSubcoreMesh(core_axis_name="c", subcore_axis_name="s")
_sc_params = pltpu.CompilerParams(use_tc_tiling_on_sc=False)


@functools.partial(
    pl.kernel,
    out_type=jax.ShapeDtypeStruct((2 * NPAD, 8), jnp.float32),
    mesh=_mesh,
    compiler_params=_sc_params,
    scratch_types=[
        pltpu.VMEM((80, CHUNK), jnp.int32),
        pltpu.VMEM((CHUNK, 8), jnp.float32),
        pltpu.VMEM((640, 8), jnp.float32),
        pltpu.VMEM_SHARED((NPAD, 8), jnp.float32),
        pltpu.SemaphoreType.DMA,
    ],
)
def _deg_k(dst_hbm, ones_hbm, half_hbm, out_hbm, dst_v, ones_v, stage_v,
           acc_s, sem):
    c = lax.axis_index("c")
    s = lax.axis_index("s")
    tid = c * NS + s
    pltpu.sync_copy(dst_hbm.at[tid], dst_v)
    pltpu.sync_copy(ones_hbm, ones_v)
    pltpu.sync_copy(half_hbm.at[pl.ds(s * 640, 640)], stage_v)
    pltpu.sync_copy(stage_v, acc_s.at[pl.ds(s * 640, 640)])
    plsc.subcore_barrier()

    def body(j, carry):
        pltpu.sync_copy(ones_v, acc_s.at[dst_v.at[j]], add=True)
        return carry

    lax.fori_loop(0, 80, body, 0)
    plsc.subcore_barrier()
    pltpu.sync_copy(acc_s.at[pl.ds(s * 640, 640)], stage_v)
    pltpu.sync_copy(stage_v, out_hbm.at[pl.ds(c * NPAD + s * 640, 640)])


def _make_prop(F):
    rows_per_tile = NPAD // NS

    @functools.partial(
        pl.kernel,
        out_type=jax.ShapeDtypeStruct((2 * NPAD, F), jnp.float32),
        mesh=_mesh,
        compiler_params=_sc_params,
        scratch_types=[
            pltpu.VMEM((160, CHUNK), jnp.int32),
            pltpu.VMEM((160, CHUNK), jnp.int32),
            pltpu.VMEM((4, CHUNK, F), jnp.float32),
            pltpu.VMEM_SHARED((NPAD, F), jnp.float32),
            pltpu.SemaphoreType.DMA((4,)),
            pltpu.SemaphoreType.DMA((4,)),
        ],
    )
    def prop(u_hbm, src_hbm, dst_hbm, out_hbm, src_v, dst_v, rows_v,
             acc_s, semg, sems):
        c = lax.axis_index("c")
        s = lax.axis_index("s")
        tid = c * NS + s
        base = s * rows_per_tile
        pltpu.sync_copy(src_hbm.at[tid], src_v)
        pltpu.sync_copy(dst_hbm.at[tid], dst_v)
        for k in range(rows_per_tile // CHUNK):
            pltpu.sync_copy(
                u_hbm.at[pl.ds(c * NPAD + base + k * CHUNK, CHUNK)],
                rows_v.at[0])
            pltpu.sync_copy(rows_v.at[0],
                            acc_s.at[pl.ds(base + k * CHUNK, CHUNK)])
        plsc.subcore_barrier()

        def gather(j, slot):
            pltpu.async_copy(u_hbm.at[src_v.at[j]], rows_v.at[slot],
                             semg.at[slot])

        def gwait(j, slot):
            pltpu.make_async_copy(u_hbm.at[src_v.at[j]], rows_v.at[slot],
                                  semg.at[slot]).wait()

        def scat(j, slot):
            pltpu.async_copy(rows_v.at[slot], acc_s.at[dst_v.at[j]],
                             sems.at[slot], add=True)

        def swait(j, slot):
            pltpu.make_async_copy(rows_v.at[slot], acc_s.at[dst_v.at[j]],
                                  sems.at[slot]).wait()

        for b in range(2):
            gather(b, b)

        def body(g, carry):
            j4 = 4 * g
            for b in range(4):
                j = j4 + b
                gwait(j, b)
                scat(j, b)
                pb = (b + 2) % 4

                @pl.when(j >= 2)
                def _():
                    swait(j - 2, pb)

                @pl.when(j + 2 < 160)
                def _():
                    gather(j + 2, pb)
            return carry

        lax.fori_loop(0, 40, body, 0)
        for j in range(158, 160):
            swait(j, j % 4)
        plsc.subcore_barrier()
        for k in range(rows_per_tile // CHUNK):
            pltpu.sync_copy(acc_s.at[pl.ds(base + k * CHUNK, CHUNK)],
                            rows_v.at[0])
            pltpu.sync_copy(
                rows_v.at[0],
                out_hbm.at[pl.ds(c * NPAD + base + k * CHUNK, CHUNK)])

    return prop


_prop64 = _make_prop(64)
_prop32 = _make_prop(32)


def _scale_body(degT_ref, x_ref, u_ref, dis_ref):
    deg = degT_ref[:, 0:1] + degT_ref[:, 1:2]
    dis = lax.rsqrt(jnp.maximum(deg, 1e-12))
    dis_ref[...] = jnp.broadcast_to(dis, (NPAD, 8))
    u_ref[0] = x_ref[:, :64] * dis
    u_ref[1] = x_ref[:, 64:] * dis


_scale_k = pl.pallas_call(
    _scale_body,
    out_shape=(
        jax.ShapeDtypeStruct((2, NPAD, 64), jnp.float32),
        jax.ShapeDtypeStruct((NPAD, 8), jnp.float32),
    ),
)


_ROWS = 256


def _mm_body(w_ref, dis_ref, W1a_ref, b1a_ref, W1b_ref, W2a_ref, b2a_ref,
             W2b_ref, o_ref):
    dis = dis_ref[:, 0:1]
    z = jnp.concatenate([w_ref[0], w_ref[1]], axis=1) * dis
    h1 = jnp.maximum(
        jnp.dot(z, W1a_ref[...], preferred_element_type=jnp.float32)
        + b1a_ref[...], 0.0)
    g1 = jnp.dot(h1, W1b_ref[...], preferred_element_type=jnp.float32)
    o_ref[0] = g1 * dis
    h2 = jnp.maximum(
        jnp.dot(z, W2a_ref[...], preferred_element_type=jnp.float32)
        + b2a_ref[...], 0.0)
    g2 = jnp.dot(h2, W2b_ref[...], preferred_element_type=jnp.float32)
    o_ref[1] = g2 * dis


_mm_k = pl.pallas_call(
    _mm_body,
    grid=(NPAD // _ROWS,),
    in_specs=[
        pl.BlockSpec((2, _ROWS, 64), lambda i: (0, i, 0)),
        pl.BlockSpec((_ROWS, 8), lambda i: (i, 0)),
        pl.BlockSpec((128, 256), lambda i: (0, 0)),
        pl.BlockSpec((1, 256), lambda i: (0, 0)),
        pl.BlockSpec((256, 32), lambda i: (0, 0)),
        pl.BlockSpec((128, 256), lambda i: (0, 0)),
        pl.BlockSpec((1, 256), lambda i: (0, 0)),
        pl.BlockSpec((256, 32), lambda i: (0, 0)),
    ],
    out_specs=pl.BlockSpec((2, _ROWS, 32), lambda i: (0, i, 0)),
    out_shape=jax.ShapeDtypeStruct((2, NPAD, 32), jnp.float32),
)


def _fin_body(w2_ref, dis_ref, b1b_ref, b2b_ref, o1_ref, o2_ref):
    dis = dis_ref[:, 0:1]
    o1_ref[...] = w2_ref[0] * dis + b1b_ref[...]
    o2_ref[...] = w2_ref[1] * dis + b2b_ref[...]


_fin_k = pl.pallas_call(
    _fin_body,
    out_shape=(
        jax.ShapeDtypeStruct((NPAD, 32), jnp.float32),
        jax.ShapeDtypeStruct((NPAD, 32), jnp.float32),
    ),
)


def kernel(x, edge_index, W1a, b1a, W1b, b1b, W2a, b2a, W2b, b2b):
    src = edge_index[0]
    dst = edge_index[1]
    pad = jnp.full((EPAD - E,), N, jnp.int32)
    srcp = jnp.concatenate([src, pad])
    dstp = jnp.concatenate([dst, pad])
    dstA = dstp.reshape(32, 80, CHUNK)
    src16 = srcp.reshape(16, 160, CHUNK)
    dst16 = dstp.reshape(16, 160, CHUNK)
    srcT = jnp.concatenate([src16, src16 + NPAD], axis=0)
    dstT = jnp.concatenate([dst16, dst16], axis=0)
    x_pad = jnp.pad(x, ((0, NPAD - N), (0, 0)))

    degpair = _deg_k(dstA, jnp.full((CHUNK, 8), 1.0, jnp.float32),
                     jnp.full((NPAD, 8), 0.5, jnp.float32))
    degT = degpair.reshape(2, NPAD, 8)[:, :, 0].T
    u, dis8 = _scale_k(degT, x_pad)
    w = _prop64(u.reshape(2 * NPAD, 64), srcT, dstT)
    u2 = _mm_k(w.reshape(2, NPAD, 64), dis8, W1a, b1a.reshape(1, -1), W1b,
               W2a, b2a.reshape(1, -1), W2b)
    w2 = _prop32(u2.reshape(2 * NPAD, 32), srcT, dstT)
    o1, o2 = _fin_k(w2.reshape(2, NPAD, 32), dis8, b1b.reshape(1, -1),
                    b2b.reshape(1, -1))
    return (o1[:N], o2[:N])


# speedup vs baseline: 20.9736x; 1.0106x over previous
import functools

import jax
import jax.numpy as jnp
from jax import lax
from jax.experimental import pallas as pl
from jax.experimental.pallas import tpu as pltpu
from jax.experimental.pallas import tpu_sc as plsc

N = 10000
NPAD = 10240
E = 320000
EPAD = 327680
NC, NS = 2, 16
CHUNK = 128

_mesh = plsc.VectorSubcoreMesh(core_axis_name="c", subcore_axis_name="s")
_sc_params = pltpu.CompilerParams(use_tc_tiling_on_sc=False)


@functools.partial(
    pl.kernel,
    out_type=jax.ShapeDtypeStruct((2 * NPAD, 8), jnp.float32),
    mesh=_mesh,
    compiler_params=_sc_params,
    scratch_types=[
        pltpu.VMEM((80, CHUNK), jnp.int32),
        pltpu.VMEM((CHUNK, 8), jnp.float32),
        pltpu.VMEM((640, 8), jnp.float32),
        pltpu.VMEM_SHARED((NPAD, 8), jnp.float32),
        pltpu.SemaphoreType.DMA,
    ],
)
def _deg_k(dst_hbm, ones_hbm, half_hbm, out_hbm, dst_v, ones_v, stage_v,
           acc_s, sem):
    c = lax.axis_index("c")
    s = lax.axis_index("s")
    tid = c * NS + s
    pltpu.sync_copy(dst_hbm.at[tid], dst_v)
    pltpu.sync_copy(ones_hbm, ones_v)
    pltpu.sync_copy(half_hbm.at[pl.ds(s * 640, 640)], stage_v)
    pltpu.sync_copy(stage_v, acc_s.at[pl.ds(s * 640, 640)])
    plsc.subcore_barrier()

    def body(j, carry):
        pltpu.sync_copy(ones_v, acc_s.at[dst_v.at[j]], add=True)
        return carry

    lax.fori_loop(0, 80, body, 0)
    plsc.subcore_barrier()
    pltpu.sync_copy(acc_s.at[pl.ds(s * 640, 640)], stage_v)
    pltpu.sync_copy(stage_v, out_hbm.at[pl.ds(c * NPAD + s * 640, 640)])


def _make_prop(F, scale_bias=False):
    rows_per_tile = NPAD // NS

    extra_in = []
    extra_scratch = []
    if scale_bias:
        extra_scratch = [
            pltpu.VMEM((rows_per_tile * 8 + 16,), jnp.float32),
            pltpu.VMEM((F,), jnp.float32),
        ]

    @functools.partial(
        pl.kernel,
        out_type=jax.ShapeDtypeStruct((2, NPAD, F), jnp.float32),
        mesh=_mesh,
        compiler_params=_sc_params,
        scratch_types=[
            pltpu.VMEM((160, CHUNK), jnp.int32),
            pltpu.VMEM((160, CHUNK), jnp.int32),
            pltpu.VMEM((4, CHUNK, F), jnp.float32),
            pltpu.VMEM_SHARED((NPAD, F), jnp.float32),
            pltpu.SemaphoreType.DMA((4,)),
            pltpu.SemaphoreType.DMA((4,)),
        ] + extra_scratch,
    )
    def prop(u_hbm, src_hbm, dst_hbm, *rest):
        if scale_bias:
            (dis_hbm, b_hbm, out_hbm, src_v, dst_v, rows_v, acc_s, semg,
             sems, dis_v, b_v) = rest
        else:
            (out_hbm, src_v, dst_v, rows_v, acc_s, semg, sems) = rest
        c = lax.axis_index("c")
        s = lax.axis_index("s")
        base = s * rows_per_tile
        uc = u_hbm.at[c]
        oc = out_hbm.at[c]
        pltpu.sync_copy(src_hbm.at[s], src_v)
        pltpu.sync_copy(dst_hbm.at[s], dst_v)

        for k in range(rows_per_tile // CHUNK):
            pltpu.sync_copy(uc.at[pl.ds(base + k * CHUNK, CHUNK)],
                            rows_v.at[0])
            pltpu.sync_copy(rows_v.at[0],
                            acc_s.at[pl.ds(base + k * CHUNK, CHUNK)])
        plsc.subcore_barrier()

        def gather(j, slot):
            pltpu.async_copy(uc.at[src_v.at[j]], rows_v.at[slot],
                             semg.at[slot])

        def gwait(j, slot):
            pltpu.make_async_copy(uc.at[src_v.at[j]], rows_v.at[slot],
                                  semg.at[slot]).wait()

        def scat(j, slot):
            pltpu.async_copy(rows_v.at[slot], acc_s.at[dst_v.at[j]],
                             sems.at[slot], add=True)

        def swait(j, slot):
            pltpu.make_async_copy(rows_v.at[slot], acc_s.at[dst_v.at[j]],
                                  sems.at[slot]).wait()

        for b in range(2):
            gather(b, b)

        def body(g, carry):
            j4 = 4 * g
            for b in range(4):
                j = j4 + b
                gwait(j, b)
                scat(j, b)
                pb = (b + 2) % 4

                @pl.when(j >= 2)
                def _():
                    swait(j - 2, pb)

                @pl.when(j + 2 < 160)
                def _():
                    gather(j + 2, pb)
            return carry

        lax.fori_loop(0, 40, body, 0)
        for j in range(158, 160):
            swait(j, j % 4)
        plsc.subcore_barrier()

        if scale_bias:
            pltpu.sync_copy(dis_hbm.at[pl.ds(base * 8, rows_per_tile * 8)],
                            dis_v.at[pl.ds(0, rows_per_tile * 8)])
            pltpu.sync_copy(b_hbm.at[c], b_v)
        for k in range(rows_per_tile // CHUNK):
            pltpu.sync_copy(acc_s.at[pl.ds(base + k * CHUNK, CHUNK)],
                            rows_v.at[0])
            if scale_bias:
                rv = rows_v.at[0]

                def rowfix(r, carry, k=k, rv=rv):
                    dsc = dis_v[pl.ds((k * CHUNK + r) * 8, 16)][0]
                    for h in range(F // 16):
                        rv[r, pl.ds(h * 16, 16)] = (
                            rv[r, pl.ds(h * 16, 16)] * dsc
                            + b_v[pl.ds(h * 16, 16)])
                    return carry

                lax.fori_loop(0, CHUNK, rowfix, 0)
            pltpu.sync_copy(rows_v.at[0],
                            oc.at[pl.ds(base + k * CHUNK, CHUNK)])

    return prop


_prop64 = _make_prop(64)
_prop32 = _make_prop(32, scale_bias=True)


def _scale_body(degT_ref, x_ref, u_ref, dis_ref):
    deg = degT_ref[:, 0:1] + degT_ref[:, 1:2]
    dis = lax.rsqrt(jnp.maximum(deg, 1e-12))
    dis_ref[...] = jnp.broadcast_to(dis, (NPAD, 8))
    u_ref[0] = x_ref[:, :64] * dis
    u_ref[1] = x_ref[:, 64:] * dis


_scale_k = pl.pallas_call(
    _scale_body,
    out_shape=(
        jax.ShapeDtypeStruct((2, NPAD, 64), jnp.float32),
        jax.ShapeDtypeStruct((NPAD, 8), jnp.float32),
    ),
)


_ROWS = 256


def _mm_body(w_ref, dis_ref, W1a_ref, b1a_ref, W1b_ref, W2a_ref, b2a_ref,
             W2b_ref, o_ref):
    dis = dis_ref[:, 0:1]
    z = jnp.concatenate([w_ref[0], w_ref[1]], axis=1) * dis
    h1 = jnp.maximum(
        jnp.dot(z, W1a_ref[...], preferred_element_type=jnp.float32)
        + b1a_ref[...], 0.0)
    g1 = jnp.dot(h1, W1b_ref[...], preferred_element_type=jnp.float32)
    o_ref[0] = g1 * dis
    h2 = jnp.maximum(
        jnp.dot(z, W2a_ref[...], preferred_element_type=jnp.float32)
        + b2a_ref[...], 0.0)
    g2 = jnp.dot(h2, W2b_ref[...], preferred_element_type=jnp.float32)
    o_ref[1] = g2 * dis


_mm_k = pl.pallas_call(
    _mm_body,
    grid=(NPAD // _ROWS,),
    in_specs=[
        pl.BlockSpec((2, _ROWS, 64), lambda i: (0, i, 0)),
        pl.BlockSpec((_ROWS, 8), lambda i: (i, 0)),
        pl.BlockSpec((128, 256), lambda i: (0, 0)),
        pl.BlockSpec((1, 256), lambda i: (0, 0)),
        pl.BlockSpec((256, 32), lambda i: (0, 0)),
        pl.BlockSpec((128, 256), lambda i: (0, 0)),
        pl.BlockSpec((1, 256), lambda i: (0, 0)),
        pl.BlockSpec((256, 32), lambda i: (0, 0)),
    ],
    out_specs=pl.BlockSpec((2, _ROWS, 32), lambda i: (0, i, 0)),
    out_shape=jax.ShapeDtypeStruct((2, NPAD, 32), jnp.float32),
)


def kernel(x, edge_index, W1a, b1a, W1b, b1b, W2a, b2a, W2b, b2b):
    src = edge_index[0]
    dst = edge_index[1]
    pad = jnp.full((EPAD - E,), N, jnp.int32)
    srcp = jnp.concatenate([src, pad])
    dstp = jnp.concatenate([dst, pad])
    dstA = dstp.reshape(32, 80, CHUNK)
    src16 = srcp.reshape(16, 160, CHUNK)
    dst16 = dstp.reshape(16, 160, CHUNK)
    x_pad = jnp.pad(x, ((0, NPAD - N), (0, 0)))

    degpair = _deg_k(dstA, jnp.full((CHUNK, 8), 1.0, jnp.float32),
                     jnp.full((NPAD, 8), 0.5, jnp.float32))
    degT = degpair.reshape(2, NPAD, 8)[:, :, 0].T
    u, dis8 = _scale_k(degT, x_pad)
    w = _prop64(u, src16, dst16)
    u2 = _mm_k(w, dis8, W1a, b1a.reshape(1, -1), W1b,
               W2a, b2a.reshape(1, -1), W2b)
    out = _prop32(u2, src16, dst16, dis8.reshape(-1), jnp.stack([b1b, b2b]))
    return (out[0, :N], out[1, :N])


# speedup vs baseline: 24.4267x; 1.1646x over previous
import functools

import jax
import jax.numpy as jnp
from jax import lax
from jax.experimental import pallas as pl
from jax.experimental.pallas import tpu as pltpu
from jax.experimental.pallas import tpu_sc as plsc

N = 10000
NPAD = 10240
E = 320000
EPAD = 327680
NC, NS = 2, 16
CHUNK = 128

_mesh = plsc.VectorSubcoreMesh(core_axis_name="c", subcore_axis_name="s")
_sc_params = pltpu.CompilerParams(use_tc_tiling_on_sc=False)


@functools.partial(
    pl.kernel,
    out_type=jax.ShapeDtypeStruct((2 * NPAD, 8), jnp.float32),
    mesh=_mesh,
    compiler_params=_sc_params,
    scratch_types=[
        pltpu.VMEM((80, CHUNK), jnp.int32),
        pltpu.VMEM((CHUNK, 8), jnp.float32),
        pltpu.VMEM((640, 8), jnp.float32),
        pltpu.VMEM_SHARED((NPAD, 8), jnp.float32),
        pltpu.SemaphoreType.DMA,
    ],
)
def _deg_k(dst_hbm, ones_hbm, half_hbm, out_hbm, dst_v, ones_v, stage_v,
           acc_s, sem):
    c = lax.axis_index("c")
    s = lax.axis_index("s")
    tid = c * NS + s
    pltpu.sync_copy(dst_hbm.at[tid], dst_v)
    pltpu.sync_copy(ones_hbm, ones_v)
    pltpu.sync_copy(half_hbm.at[pl.ds(s * 640, 640)], stage_v)
    pltpu.sync_copy(stage_v, acc_s.at[pl.ds(s * 640, 640)])
    plsc.subcore_barrier()

    def body(j, carry):
        pltpu.sync_copy(ones_v, acc_s.at[dst_v.at[j]], add=True)
        return carry

    lax.fori_loop(0, 80, body, 0)
    plsc.subcore_barrier()
    pltpu.sync_copy(acc_s.at[pl.ds(s * 640, 640)], stage_v)
    pltpu.sync_copy(stage_v, out_hbm.at[pl.ds(c * NPAD + s * 640, 640)])


def _make_prop(F, scale_bias=False, src_spmem=False):
    rows_per_tile = NPAD // NS

    extra_scratch = []
    if scale_bias:
        extra_scratch += [
            pltpu.VMEM((rows_per_tile * 8 + 16,), jnp.float32),
            pltpu.VMEM((F,), jnp.float32),
        ]
    if src_spmem:
        extra_scratch += [pltpu.VMEM_SHARED((NPAD, F), jnp.float32)]

    @functools.partial(
        pl.kernel,
        out_type=jax.ShapeDtypeStruct((2, NPAD, F), jnp.float32),
        mesh=_mesh,
        compiler_params=_sc_params,
        scratch_types=[
            pltpu.VMEM((160, CHUNK), jnp.int32),
            pltpu.VMEM((160, CHUNK), jnp.int32),
            pltpu.VMEM((4, CHUNK, F), jnp.float32),
            pltpu.VMEM_SHARED((NPAD, F), jnp.float32),
            pltpu.SemaphoreType.DMA((4,)),
            pltpu.SemaphoreType.DMA((4,)),
        ] + extra_scratch,
    )
    def prop(u_hbm, src_hbm, dst_hbm, *rest):
        rest = list(rest)
        if scale_bias:
            dis_hbm, b_hbm = rest[:2]
            rest = rest[2:]
        out_hbm, src_v, dst_v, rows_v, acc_s, semg, sems = rest[:7]
        rest = rest[7:]
        if scale_bias:
            dis_v, b_v = rest[:2]
            rest = rest[2:]
        if src_spmem:
            u_s = rest[0]
        c = lax.axis_index("c")
        s = lax.axis_index("s")
        base = s * rows_per_tile
        uc = u_hbm.at[c]
        oc = out_hbm.at[c]
        pltpu.sync_copy(src_hbm.at[s], src_v)
        pltpu.sync_copy(dst_hbm.at[s], dst_v)

        for k in range(rows_per_tile // CHUNK):
            pltpu.sync_copy(uc.at[pl.ds(base + k * CHUNK, CHUNK)],
                            rows_v.at[0])
            pltpu.sync_copy(rows_v.at[0],
                            acc_s.at[pl.ds(base + k * CHUNK, CHUNK)])
            if src_spmem:
                pltpu.sync_copy(rows_v.at[0],
                                u_s.at[pl.ds(base + k * CHUNK, CHUNK)])
        plsc.subcore_barrier()
        usrc = u_s if src_spmem else uc

        def gather(j, slot):
            pltpu.async_copy(usrc.at[src_v.at[j]], rows_v.at[slot],
                             semg.at[slot])

        def gwait(j, slot):
            pltpu.make_async_copy(usrc.at[src_v.at[j]], rows_v.at[slot],
                                  semg.at[slot]).wait()

        def scat(j, slot):
            pltpu.async_copy(rows_v.at[slot], acc_s.at[dst_v.at[j]],
                             sems.at[slot], add=True)

        def swait(j, slot):
            pltpu.make_async_copy(rows_v.at[slot], acc_s.at[dst_v.at[j]],
                                  sems.at[slot]).wait()

        for b in range(2):
            gather(b, b)

        def body(g, carry):
            j4 = 4 * g
            for b in range(4):
                j = j4 + b
                gwait(j, b)
                scat(j, b)
                pb = (b + 2) % 4

                @pl.when(j >= 2)
                def _():
                    swait(j - 2, pb)

                @pl.when(j + 2 < 160)
                def _():
                    gather(j + 2, pb)
            return carry

        lax.fori_loop(0, 40, body, 0)
        for j in range(158, 160):
            swait(j, j % 4)
        plsc.subcore_barrier()

        if scale_bias:
            pltpu.sync_copy(dis_hbm.at[pl.ds(base * 8, rows_per_tile * 8)],
                            dis_v.at[pl.ds(0, rows_per_tile * 8)])
            pltpu.sync_copy(b_hbm.at[c], b_v)
        for k in range(rows_per_tile // CHUNK):
            pltpu.sync_copy(acc_s.at[pl.ds(base + k * CHUNK, CHUNK)],
                            rows_v.at[0])
            if scale_bias:
                rv = rows_v.at[0]

                def rowfix(r, carry, k=k, rv=rv):
                    dsc = dis_v[pl.ds((k * CHUNK + r) * 8, 16)][0]
                    for h in range(F // 16):
                        rv[r, pl.ds(h * 16, 16)] = (
                            rv[r, pl.ds(h * 16, 16)] * dsc
                            + b_v[pl.ds(h * 16, 16)])
                    return carry

                lax.fori_loop(0, CHUNK, rowfix, 0)
            pltpu.sync_copy(rows_v.at[0],
                            oc.at[pl.ds(base + k * CHUNK, CHUNK)])

    return prop


_prop64 = _make_prop(64)
_prop32 = _make_prop(32, scale_bias=True, src_spmem=True)


def _scale_body(degT_ref, x_ref, u_ref, dis_ref):
    deg = degT_ref[:, 0:1] + degT_ref[:, 1:2]
    dis = lax.rsqrt(jnp.maximum(deg, 1e-12))
    dis_ref[...] = jnp.broadcast_to(dis, (NPAD, 8))
    u_ref[0] = x_ref[:, :64] * dis
    u_ref[1] = x_ref[:, 64:] * dis


_scale_k = pl.pallas_call(
    _scale_body,
    out_shape=(
        jax.ShapeDtypeStruct((2, NPAD, 64), jnp.float32),
        jax.ShapeDtypeStruct((NPAD, 8), jnp.float32),
    ),
)


_ROWS = 256


def _mm_body(w_ref, dis_ref, W1a_ref, b1a_ref, W1b_ref, W2a_ref, b2a_ref,
             W2b_ref, o_ref):
    dis = dis_ref[:, 0:1]
    z = jnp.concatenate([w_ref[0], w_ref[1]], axis=1) * dis
    h1 = jnp.maximum(
        jnp.dot(z, W1a_ref[...], preferred_element_type=jnp.float32)
        + b1a_ref[...], 0.0)
    g1 = jnp.dot(h1, W1b_ref[...], preferred_element_type=jnp.float32)
    o_ref[0] = g1 * dis
    h2 = jnp.maximum(
        jnp.dot(z, W2a_ref[...], preferred_element_type=jnp.float32)
        + b2a_ref[...], 0.0)
    g2 = jnp.dot(h2, W2b_ref[...], preferred_element_type=jnp.float32)
    o_ref[1] = g2 * dis


_mm_k = pl.pallas_call(
    _mm_body,
    grid=(NPAD // _ROWS,),
    in_specs=[
        pl.BlockSpec((2, _ROWS, 64), lambda i: (0, i, 0)),
        pl.BlockSpec((_ROWS, 8), lambda i: (i, 0)),
        pl.BlockSpec((128, 256), lambda i: (0, 0)),
        pl.BlockSpec((1, 256), lambda i: (0, 0)),
        pl.BlockSpec((256, 32), lambda i: (0, 0)),
        pl.BlockSpec((128, 256), lambda i: (0, 0)),
        pl.BlockSpec((1, 256), lambda i: (0, 0)),
        pl.BlockSpec((256, 32), lambda i: (0, 0)),
    ],
    out_specs=pl.BlockSpec((2, _ROWS, 32), lambda i: (0, i, 0)),
    out_shape=jax.ShapeDtypeStruct((2, NPAD, 32), jnp.float32),
)


def kernel(x, edge_index, W1a, b1a, W1b, b1b, W2a, b2a, W2b, b2b):
    src = edge_index[0]
    dst = edge_index[1]
    pad = jnp.full((EPAD - E,), N, jnp.int32)
    srcp = jnp.concatenate([src, pad])
    dstp = jnp.concatenate([dst, pad])
    dstA = dstp.reshape(32, 80, CHUNK)
    src16 = srcp.reshape(16, 160, CHUNK)
    dst16 = dstp.reshape(16, 160, CHUNK)
    x_pad = jnp.pad(x, ((0, NPAD - N), (0, 0)))

    degpair = _deg_k(dstA, jnp.full((CHUNK, 8), 1.0, jnp.float32),
                     jnp.full((NPAD, 8), 0.5, jnp.float32))
    degT = degpair.reshape(2, NPAD, 8)[:, :, 0].T
    u, dis8 = _scale_k(degT, x_pad)
    w = _prop64(u, src16, dst16)
    u2 = _mm_k(w, dis8, W1a, b1a.reshape(1, -1), W1b,
               W2a, b2a.reshape(1, -1), W2b)
    out = _prop32(u2, src16, dst16, dis8.reshape(-1), jnp.stack([b1b, b2b]))
    return (out[0, :N], out[1, :N])


# speedup vs baseline: 35.2134x; 1.4416x over previous
import functools

import jax
import jax.numpy as jnp
from jax import lax
from jax.experimental import pallas as pl
from jax.experimental.pallas import tpu as pltpu
from jax.experimental.pallas import tpu_sc as plsc

N = 10000
NPAD = 10240
E = 320000
EPAD = 327680
NC, NS = 2, 16
CHUNK = 128

_mesh = plsc.VectorSubcoreMesh(core_axis_name="c", subcore_axis_name="s")
_sc_params = pltpu.CompilerParams(use_tc_tiling_on_sc=False)


@functools.partial(
    pl.kernel,
    out_type=jax.ShapeDtypeStruct((2 * NPAD, 8), jnp.float32),
    mesh=_mesh,
    compiler_params=_sc_params,
    scratch_types=[
        pltpu.VMEM((80, CHUNK), jnp.int32),
        pltpu.VMEM((CHUNK, 8), jnp.float32),
        pltpu.VMEM((640, 8), jnp.float32),
        pltpu.VMEM_SHARED((NPAD, 8), jnp.float32),
        pltpu.SemaphoreType.DMA,
    ],
)
def _deg_k(dst_hbm, ones_hbm, half_hbm, out_hbm, dst_v, ones_v, stage_v,
           acc_s, sem):
    c = lax.axis_index("c")
    s = lax.axis_index("s")
    tid = c * NS + s
    pltpu.sync_copy(dst_hbm.at[tid], dst_v)
    pltpu.sync_copy(ones_hbm, ones_v)
    pltpu.sync_copy(half_hbm.at[pl.ds(s * 640, 640)], stage_v)
    pltpu.sync_copy(stage_v, acc_s.at[pl.ds(s * 640, 640)])
    plsc.subcore_barrier()

    def body(j, carry):
        pltpu.sync_copy(ones_v, acc_s.at[dst_v.at[j]], add=True)
        return carry

    lax.fori_loop(0, 80, body, 0)
    plsc.subcore_barrier()
    pltpu.sync_copy(acc_s.at[pl.ds(s * 640, 640)], stage_v)
    pltpu.sync_copy(stage_v, out_hbm.at[pl.ds(c * NPAD + s * 640, 640)])


def _make_prop(F, scale_bias=False, src_spmem=False, nslots=4, nphases=1):
    rows_per_tile = NPAD // NS
    cpp = 160 // nphases
    ahead = nslots - 2

    extra_scratch = []
    if scale_bias:
        extra_scratch += [
            pltpu.VMEM((rows_per_tile * 8 + 16,), jnp.float32),
            pltpu.VMEM((F,), jnp.float32),
        ]
    if src_spmem:
        extra_scratch += [pltpu.VMEM_SHARED((NPAD, F), jnp.float32)]

    @functools.partial(
        pl.kernel,
        out_type=jax.ShapeDtypeStruct((2, NPAD, F), jnp.float32),
        mesh=_mesh,
        compiler_params=_sc_params,
        scratch_types=[
            pltpu.VMEM((cpp, CHUNK), jnp.int32),
            pltpu.VMEM((cpp, CHUNK), jnp.int32),
            pltpu.VMEM((nslots, CHUNK, F), jnp.float32),
            pltpu.VMEM_SHARED((NPAD, F), jnp.float32),
            pltpu.SemaphoreType.DMA((nslots,)),
            pltpu.SemaphoreType.DMA((nslots,)),
        ] + extra_scratch,
    )
    def prop(u_hbm, src_hbm, dst_hbm, *rest):
        rest = list(rest)
        if scale_bias:
            dis_hbm, b_hbm = rest[:2]
            rest = rest[2:]
        out_hbm, src_v, dst_v, rows_v, acc_s, semg, sems = rest[:7]
        rest = rest[7:]
        if scale_bias:
            dis_v, b_v = rest[:2]
            rest = rest[2:]
        if src_spmem:
            u_s = rest[0]
        c = lax.axis_index("c")
        s = lax.axis_index("s")
        base = s * rows_per_tile
        uc = u_hbm.at[c]
        oc = out_hbm.at[c]

        for k in range(rows_per_tile // CHUNK):
            pltpu.sync_copy(uc.at[pl.ds(base + k * CHUNK, CHUNK)],
                            rows_v.at[0])
            pltpu.sync_copy(rows_v.at[0],
                            acc_s.at[pl.ds(base + k * CHUNK, CHUNK)])
            if src_spmem:
                pltpu.sync_copy(rows_v.at[0],
                                u_s.at[pl.ds(base + k * CHUNK, CHUNK)])
        plsc.subcore_barrier()
        usrc = u_s if src_spmem else uc

        def gather(j, slot):
            pltpu.async_copy(usrc.at[src_v.at[j]], rows_v.at[slot],
                             semg.at[slot])

        def gwait(j, slot):
            pltpu.make_async_copy(usrc.at[src_v.at[j]], rows_v.at[slot],
                                  semg.at[slot]).wait()

        def scat(j, slot):
            pltpu.async_copy(rows_v.at[slot], acc_s.at[dst_v.at[j]],
                             sems.at[slot], add=True)

        def swait(j, slot):
            pltpu.make_async_copy(rows_v.at[slot], acc_s.at[dst_v.at[j]],
                                  sems.at[slot]).wait()

        def step(j, b):
            gwait(j, b)
            scat(j, b)
            pb = (b + ahead) % nslots
            if isinstance(j, int):
                if j + ahead - nslots >= 0:
                    swait(j + ahead - nslots, pb)
                if j + ahead < cpp:
                    gather(j + ahead, pb)
            else:
                @pl.when(j + ahead - nslots >= 0)
                def _():
                    swait(j + ahead - nslots, pb)

                @pl.when(j + ahead < cpp)
                def _():
                    gather(j + ahead, pb)

        for phase in range(nphases):
            pltpu.sync_copy(src_hbm.at[s].at[pl.ds(phase * cpp, cpp)], src_v)
            pltpu.sync_copy(dst_hbm.at[s].at[pl.ds(phase * cpp, cpp)], dst_v)
            for b in range(ahead):
                gather(b, b)

            def body(g, carry):
                jn = nslots * g
                for b in range(nslots):
                    step(jn + b, b)
                return carry

            n_grp = cpp // nslots
            lax.fori_loop(0, n_grp, body, 0)
            for j in range(n_grp * nslots, cpp):
                step(j, j % nslots)
            for j in range(cpp - nslots + ahead, cpp):
                swait(j, j % nslots)
        plsc.subcore_barrier()

        if scale_bias:
            pltpu.sync_copy(dis_hbm.at[pl.ds(base * 8, rows_per_tile * 8)],
                            dis_v.at[pl.ds(0, rows_per_tile * 8)])
            pltpu.sync_copy(b_hbm.at[c], b_v)
        for k in range(rows_per_tile // CHUNK):
            pltpu.sync_copy(acc_s.at[pl.ds(base + k * CHUNK, CHUNK)],
                            rows_v.at[0])
            if scale_bias:
                rv = rows_v.at[0]

                def rowfix(r, carry, k=k, rv=rv):
                    dsc = dis_v[pl.ds((k * CHUNK + r) * 8, 16)][0]
                    for h in range(F // 16):
                        rv[r, pl.ds(h * 16, 16)] = (
                            rv[r, pl.ds(h * 16, 16)] * dsc
                            + b_v[pl.ds(h * 16, 16)])
                    return carry

                lax.fori_loop(0, CHUNK, rowfix, 0)
            pltpu.sync_copy(rows_v.at[0],
                            oc.at[pl.ds(base + k * CHUNK, CHUNK)])

    return prop


_prop64 = _make_prop(64, src_spmem=True, nslots=3, nphases=2)
_prop32 = _make_prop(32, scale_bias=True, src_spmem=True)


def _scale_body(degT_ref, x_ref, u_ref, dis_ref):
    deg = degT_ref[:, 0:1] + degT_ref[:, 1:2]
    dis = lax.rsqrt(jnp.maximum(deg, 1e-12))
    dis_ref[...] = jnp.broadcast_to(dis, (NPAD, 8))
    u_ref[0] = x_ref[:, :64] * dis
    u_ref[1] = x_ref[:, 64:] * dis


_scale_k = pl.pallas_call(
    _scale_body,
    out_shape=(
        jax.ShapeDtypeStruct((2, NPAD, 64), jnp.float32),
        jax.ShapeDtypeStruct((NPAD, 8), jnp.float32),
    ),
)


_ROWS = 256


def _mm_body(w_ref, dis_ref, W1a_ref, b1a_ref, W1b_ref, W2a_ref, b2a_ref,
             W2b_ref, o_ref):
    dis = dis_ref[:, 0:1]
    z = jnp.concatenate([w_ref[0], w_ref[1]], axis=1) * dis
    h1 = jnp.maximum(
        jnp.dot(z, W1a_ref[...], preferred_element_type=jnp.float32)
        + b1a_ref[...], 0.0)
    g1 = jnp.dot(h1, W1b_ref[...], preferred_element_type=jnp.float32)
    o_ref[0] = g1 * dis
    h2 = jnp.maximum(
        jnp.dot(z, W2a_ref[...], preferred_element_type=jnp.float32)
        + b2a_ref[...], 0.0)
    g2 = jnp.dot(h2, W2b_ref[...], preferred_element_type=jnp.float32)
    o_ref[1] = g2 * dis


_mm_k = pl.pallas_call(
    _mm_body,
    grid=(NPAD // _ROWS,),
    in_specs=[
        pl.BlockSpec((2, _ROWS, 64), lambda i: (0, i, 0)),
        pl.BlockSpec((_ROWS, 8), lambda i: (i, 0)),
        pl.BlockSpec((128, 256), lambda i: (0, 0)),
        pl.BlockSpec((1, 256), lambda i: (0, 0)),
        pl.BlockSpec((256, 32), lambda i: (0, 0)),
        pl.BlockSpec((128, 256), lambda i: (0, 0)),
        pl.BlockSpec((1, 256), lambda i: (0, 0)),
        pl.BlockSpec((256, 32), lambda i: (0, 0)),
    ],
    out_specs=pl.BlockSpec((2, _ROWS, 32), lambda i: (0, i, 0)),
    out_shape=jax.ShapeDtypeStruct((2, NPAD, 32), jnp.float32),
)


def kernel(x, edge_index, W1a, b1a, W1b, b1b, W2a, b2a, W2b, b2b):
    src = edge_index[0]
    dst = edge_index[1]
    pad = jnp.full((EPAD - E,), N, jnp.int32)
    srcp = jnp.concatenate([src, pad])
    dstp = jnp.concatenate([dst, pad])
    dstA = dstp.reshape(32, 80, CHUNK)
    src16 = srcp.reshape(16, 160, CHUNK)
    dst16 = dstp.reshape(16, 160, CHUNK)
    x_pad = jnp.pad(x, ((0, NPAD - N), (0, 0)))

    degpair = _deg_k(dstA, jnp.full((CHUNK, 8), 1.0, jnp.float32),
                     jnp.full((NPAD, 8), 0.5, jnp.float32))
    degT = degpair.reshape(2, NPAD, 8)[:, :, 0].T
    u, dis8 = _scale_k(degT, x_pad)
    w = _prop64(u, src16, dst16)
    u2 = _mm_k(w, dis8, W1a, b1a.reshape(1, -1), W1b,
               W2a, b2a.reshape(1, -1), W2b)
    out = _prop32(u2, src16, dst16, dis8.reshape(-1), jnp.stack([b1b, b2b]))
    return (out[0, :N], out[1, :N])


# speedup vs baseline: 40.7818x; 1.1581x over previous
import functools

import jax
import jax.numpy as jnp
from jax import lax
from jax.experimental import pallas as pl
from jax.experimental.pallas import tpu as pltpu
from jax.experimental.pallas import tpu_sc as plsc

N = 10000
NPAD = 10240
E = 320000
EPAD = 327680
NC, NS = 2, 16
CHUNK = 128

_mesh = plsc.VectorSubcoreMesh(core_axis_name="c", subcore_axis_name="s")
_sc_params = pltpu.CompilerParams(use_tc_tiling_on_sc=False)


@functools.partial(
    pl.kernel,
    out_type=jax.ShapeDtypeStruct((2 * NPAD, 8), jnp.float32),
    mesh=_mesh,
    compiler_params=_sc_params,
    scratch_types=[
        pltpu.VMEM((80, CHUNK), jnp.int32),
        pltpu.VMEM((CHUNK, 8), jnp.float32),
        pltpu.VMEM((640, 8), jnp.float32),
        pltpu.VMEM_SHARED((NPAD, 8), jnp.float32),
        pltpu.SemaphoreType.DMA,
    ],
)
def _deg_k(dst_hbm, ones_hbm, half_hbm, out_hbm, dst_v, ones_v, stage_v,
           acc_s, sem):
    c = lax.axis_index("c")
    s = lax.axis_index("s")
    tid = c * NS + s
    pltpu.sync_copy(dst_hbm.at[tid], dst_v)
    pltpu.sync_copy(ones_hbm, ones_v)
    pltpu.sync_copy(half_hbm.at[pl.ds(s * 640, 640)], stage_v)
    pltpu.sync_copy(stage_v, acc_s.at[pl.ds(s * 640, 640)])
    plsc.subcore_barrier()

    def body(j, carry):
        pltpu.sync_copy(ones_v, acc_s.at[dst_v.at[j]], add=True)
        return carry

    lax.fori_loop(0, 80, body, 0)
    plsc.subcore_barrier()
    pltpu.sync_copy(acc_s.at[pl.ds(s * 640, 640)], stage_v)
    pltpu.sync_copy(stage_v, out_hbm.at[pl.ds(c * NPAD + s * 640, 640)])


def _make_prop(F, newton=False, stage_scale=False, out_scale=False,
               bias=False, nslots=4, nphases=1):
    rows_per_tile = NPAD // NS
    rt8 = rows_per_tile * 8
    cpp = 160 // nphases
    ahead = nslots - 2
    needs_dis = newton or stage_scale or out_scale

    extra_scratch = [pltpu.VMEM_SHARED((NPAD, F), jnp.float32)]
    if needs_dis:
        extra_scratch += [pltpu.VMEM((rt8 + 16,), jnp.float32)]
    if newton:
        extra_scratch += [pltpu.VMEM((rt8 + 16,), jnp.float32)]
    if bias:
        extra_scratch += [pltpu.VMEM((F,), jnp.float32)]

    out_type = jax.ShapeDtypeStruct((2, NPAD, F), jnp.float32)
    if newton:
        out_type = (out_type, jax.ShapeDtypeStruct((NPAD * 8,), jnp.float32))

    @functools.partial(
        pl.kernel,
        out_type=out_type,
        mesh=_mesh,
        compiler_params=_sc_params,
        scratch_types=[
            pltpu.VMEM((cpp, CHUNK), jnp.int32),
            pltpu.VMEM((cpp, CHUNK), jnp.int32),
            pltpu.VMEM((nslots, CHUNK, F), jnp.float32),
            pltpu.VMEM_SHARED((NPAD, F), jnp.float32),
            pltpu.SemaphoreType.DMA((nslots,)),
            pltpu.SemaphoreType.DMA((nslots,)),
        ] + extra_scratch,
    )
    def prop(u_hbm, src_hbm, dst_hbm, *rest):
        rest = list(rest)
        deg_hbm = dis_hbm = b_hbm = dis8_out = None
        if newton:
            deg_hbm = rest.pop(0)
        elif needs_dis:
            dis_hbm = rest.pop(0)
        if bias:
            b_hbm = rest.pop(0)
        out_hbm = rest.pop(0)
        if newton:
            dis8_out = rest.pop(0)
        src_v, dst_v, rows_v, acc_s, semg, sems, u_s = rest[:7]
        rest = rest[7:]
        dis_v = deg2_v = b_v = None
        if needs_dis:
            dis_v = rest.pop(0)
        if newton:
            deg2_v = rest.pop(0)
        if bias:
            b_v = rest.pop(0)

        c = lax.axis_index("c")
        s = lax.axis_index("s")
        base = s * rows_per_tile
        uc = u_hbm.at[c]
        oc = out_hbm.at[c]

        if newton:
            pltpu.sync_copy(deg_hbm.at[pl.ds(base * 8, rt8)],
                            dis_v.at[pl.ds(0, rt8)])
            pltpu.sync_copy(deg_hbm.at[pl.ds((NPAD + base) * 8, rt8)],
                            deg2_v.at[pl.ds(0, rt8)])

            def nbody(i, carry):
                t = (dis_v[pl.ds(i * 16, 16)] + deg2_v[pl.ds(i * 16, 16)])
                y = jnp.full((16,), 2.0 ** -9.5, jnp.float32)
                for k in range(8, -1, -1):
                    y = jnp.where(t < 4.0 ** (k + 1),
                                  jnp.full((16,), 2.0 ** -(k + 0.5),
                                           jnp.float32), y)
                for _ in range(6):
                    y = y * (1.5 - 0.5 * t * y * y)
                dis_v[pl.ds(i * 16, 16)] = y
                return carry

            lax.fori_loop(0, rt8 // 16, nbody, 0)
            pltpu.sync_copy(dis_v.at[pl.ds(0, rt8)],
                            dis8_out.at[pl.ds(base * 8, rt8)])
        elif needs_dis:
            pltpu.sync_copy(dis_hbm.at[pl.ds(base * 8, rt8)],
                            dis_v.at[pl.ds(0, rt8)])
        if bias:
            pltpu.sync_copy(b_hbm.at[c], b_v)

        def rowscale(k, with_bias):
            rv = rows_v.at[0]

            def rowfix(r, carry):
                dsc = dis_v[pl.ds((k * CHUNK + r) * 8, 16)][0]
                for h in range(F // 16):
                    v = rv[r, pl.ds(h * 16, 16)] * dsc
                    if with_bias:
                        v = v + b_v[pl.ds(h * 16, 16)]
                    rv[r, pl.ds(h * 16, 16)] = v
                return carry

            lax.fori_loop(0, CHUNK, rowfix, 0)

        for k in range(rows_per_tile // CHUNK):
            pltpu.sync_copy(uc.at[pl.ds(base + k * CHUNK, CHUNK)],
                            rows_v.at[0])
            if stage_scale:
                rowscale(k, False)
            pltpu.sync_copy(rows_v.at[0],
                            acc_s.at[pl.ds(base + k * CHUNK, CHUNK)])
            pltpu.sync_copy(rows_v.at[0],
                            u_s.at[pl.ds(base + k * CHUNK, CHUNK)])
        plsc.subcore_barrier()

        def gather(j, slot):
            pltpu.async_copy(u_s.at[src_v.at[j]], rows_v.at[slot],
                             semg.at[slot])

        def gwait(j, slot):
            pltpu.make_async_copy(u_s.at[src_v.at[j]], rows_v.at[slot],
                                  semg.at[slot]).wait()

        def scat(j, slot):
            pltpu.async_copy(rows_v.at[slot], acc_s.at[dst_v.at[j]],
                             sems.at[slot], add=True)

        def swait(j, slot):
            pltpu.make_async_copy(rows_v.at[slot], acc_s.at[dst_v.at[j]],
                                  sems.at[slot]).wait()

        def step(j, b):
            gwait(j, b)
            scat(j, b)
            pb = (b + ahead) % nslots
            if isinstance(j, int):
                if j + ahead - nslots >= 0:
                    swait(j + ahead - nslots, pb)
                if j + ahead < cpp:
                    gather(j + ahead, pb)
            else:
                @pl.when(j + ahead - nslots >= 0)
                def _():
                    swait(j + ahead - nslots, pb)

                @pl.when(j + ahead < cpp)
                def _():
                    gather(j + ahead, pb)

        for phase in range(nphases):
            pltpu.sync_copy(src_hbm.at[s].at[pl.ds(phase * cpp, cpp)], src_v)
            pltpu.sync_copy(dst_hbm.at[s].at[pl.ds(phase * cpp, cpp)], dst_v)
            for b in range(ahead):
                gather(b, b)

            def body(g, carry):
                jn = nslots * g
                for b in range(nslots):
                    step(jn + b, b)
                return carry

            n_grp = cpp // nslots
            lax.fori_loop(0, n_grp, body, 0)
            for j in range(n_grp * nslots, cpp):
                step(j, j % nslots)
            for j in range(cpp - nslots + ahead, cpp):
                swait(j, j % nslots)
        plsc.subcore_barrier()

        for k in range(rows_per_tile // CHUNK):
            pltpu.sync_copy(acc_s.at[pl.ds(base + k * CHUNK, CHUNK)],
                            rows_v.at[0])
            if out_scale:
                rowscale(k, bias)
            pltpu.sync_copy(rows_v.at[0],
                            oc.at[pl.ds(base + k * CHUNK, CHUNK)])

    return prop


_prop64 = _make_prop(64, newton=True, stage_scale=True, out_scale=True,
                     nslots=3, nphases=4)
_prop32 = _make_prop(32, stage_scale=True, out_scale=True, bias=True,
                     nslots=4, nphases=1)


_ROWS = 256


def _mm_body(w_ref, W1a_ref, b1a_ref, W1b_ref, W2a_ref, b2a_ref,
             W2b_ref, o_ref):
    z = jnp.concatenate([w_ref[0], w_ref[1]], axis=1)
    h1 = jnp.maximum(
        jnp.dot(z, W1a_ref[...], preferred_element_type=jnp.float32)
        + b1a_ref[...], 0.0)
    o_ref[0] = jnp.dot(h1, W1b_ref[...], preferred_element_type=jnp.float32)
    h2 = jnp.maximum(
        jnp.dot(z, W2a_ref[...], preferred_element_type=jnp.float32)
        + b2a_ref[...], 0.0)
    o_ref[1] = jnp.dot(h2, W2b_ref[...], preferred_element_type=jnp.float32)


_mm_k = pl.pallas_call(
    _mm_body,
    grid=(NPAD // _ROWS,),
    in_specs=[
        pl.BlockSpec((2, _ROWS, 64), lambda i: (0, i, 0)),
        pl.BlockSpec((128, 256), lambda i: (0, 0)),
        pl.BlockSpec((1, 256), lambda i: (0, 0)),
        pl.BlockSpec((256, 32), lambda i: (0, 0)),
        pl.BlockSpec((128, 256), lambda i: (0, 0)),
        pl.BlockSpec((1, 256), lambda i: (0, 0)),
        pl.BlockSpec((256, 32), lambda i: (0, 0)),
    ],
    out_specs=pl.BlockSpec((2, _ROWS, 32), lambda i: (0, i, 0)),
    out_shape=jax.ShapeDtypeStruct((2, NPAD, 32), jnp.float32),
)


def kernel(x, edge_index, W1a, b1a, W1b, b1b, W2a, b2a, W2b, b2b):
    src = edge_index[0]
    dst = edge_index[1]
    pad = jnp.full((EPAD - E,), N, jnp.int32)
    srcp = jnp.concatenate([src, pad])
    dstp = jnp.concatenate([dst, pad])
    dstA = dstp.reshape(32, 80, CHUNK)
    src16 = srcp.reshape(16, 160, CHUNK)
    dst16 = dstp.reshape(16, 160, CHUNK)
    x_pad = jnp.pad(x, ((0, NPAD - N), (0, 0)))

    degpair = _deg_k(dstA, jnp.full((CHUNK, 8), 1.0, jnp.float32),
                     jnp.full((NPAD, 8), 0.5, jnp.float32))
    xs = jnp.stack([x_pad[:, :64], x_pad[:, 64:]])
    z, dis8 = _prop64(xs, src16, dst16, degpair.reshape(-1))
    u2 = _mm_k(z, W1a, b1a.reshape(1, -1), W1b,
               W2a, b2a.reshape(1, -1), W2b)
    out = _prop32(u2, src16, dst16, dis8, jnp.stack([b1b, b2b]))
    return (out[0, :N], out[1, :N])


# speedup vs baseline: 40.9589x; 1.0043x over previous
import functools

import jax
import jax.numpy as jnp
from jax import lax
from jax.experimental import pallas as pl
from jax.experimental.pallas import tpu as pltpu
from jax.experimental.pallas import tpu_sc as plsc

N = 10000
NPAD = 10240
E = 320000
EPAD = 327680
NC, NS = 2, 16
CHUNK = 128

_mesh = plsc.VectorSubcoreMesh(core_axis_name="c", subcore_axis_name="s")
_sc_params = pltpu.CompilerParams(use_tc_tiling_on_sc=False)


@functools.partial(
    pl.kernel,
    out_type=jax.ShapeDtypeStruct((2 * NPAD, 8), jnp.float32),
    mesh=_mesh,
    compiler_params=_sc_params,
    scratch_types=[
        pltpu.VMEM((80, CHUNK), jnp.int32),
        pltpu.VMEM((CHUNK, 8), jnp.float32),
        pltpu.VMEM((640, 8), jnp.float32),
        pltpu.VMEM_SHARED((NPAD, 8), jnp.float32),
        pltpu.SemaphoreType.DMA((4,)),
    ],
)
def _deg_k(dst_hbm, ones_hbm, half_hbm, out_hbm, dst_v, ones_v, stage_v,
           acc_s, sem):
    c = lax.axis_index("c")
    s = lax.axis_index("s")
    tid = c * NS + s
    pltpu.sync_copy(dst_hbm.at[tid], dst_v)
    pltpu.sync_copy(ones_hbm, ones_v)
    pltpu.sync_copy(half_hbm.at[pl.ds(s * 640, 640)], stage_v)
    pltpu.sync_copy(stage_v, acc_s.at[pl.ds(s * 640, 640)])
    plsc.subcore_barrier()

    def body(g, carry):
        for b in range(4):
            j = 4 * g + b

            @pl.when(j >= 4)
            def _():
                pltpu.make_async_copy(ones_v, acc_s.at[dst_v.at[j - 4]],
                                      sem.at[b]).wait()

            pltpu.async_copy(ones_v, acc_s.at[dst_v.at[j]], sem.at[b],
                             add=True)
        return carry

    lax.fori_loop(0, 20, body, 0)
    for j in range(76, 80):
        pltpu.make_async_copy(ones_v, acc_s.at[dst_v.at[j]],
                              sem.at[j % 4]).wait()
    plsc.subcore_barrier()
    pltpu.sync_copy(acc_s.at[pl.ds(s * 640, 640)], stage_v)
    pltpu.sync_copy(stage_v, out_hbm.at[pl.ds(c * NPAD + s * 640, 640)])


def _make_prop(F, newton=False, stage_scale=False, out_scale=False,
               bias=False, nslots=4, nphases=1):
    rows_per_tile = NPAD // NS
    rt8 = rows_per_tile * 8
    cpp = 160 // nphases
    ahead = nslots - 2
    needs_dis = newton or stage_scale or out_scale

    extra_scratch = [pltpu.VMEM_SHARED((NPAD, F), jnp.float32)]
    if needs_dis:
        extra_scratch += [pltpu.VMEM((rt8 + 16,), jnp.float32)]
    if newton:
        extra_scratch += [pltpu.VMEM((rt8 + 16,), jnp.float32)]
    if bias:
        extra_scratch += [pltpu.VMEM((F,), jnp.float32)]

    out_type = jax.ShapeDtypeStruct((2, NPAD, F), jnp.float32)
    if newton:
        out_type = (out_type, jax.ShapeDtypeStruct((NPAD * 8,), jnp.float32))

    @functools.partial(
        pl.kernel,
        out_type=out_type,
        mesh=_mesh,
        compiler_params=_sc_params,
        scratch_types=[
            pltpu.VMEM((cpp, CHUNK), jnp.int32),
            pltpu.VMEM((cpp, CHUNK), jnp.int32),
            pltpu.VMEM((nslots, CHUNK, F), jnp.float32),
            pltpu.VMEM_SHARED((NPAD, F), jnp.float32),
            pltpu.SemaphoreType.DMA((nslots,)),
            pltpu.SemaphoreType.DMA((nslots,)),
        ] + extra_scratch,
    )
    def prop(u_hbm, src_hbm, dst_hbm, *rest):
        rest = list(rest)
        deg_hbm = dis_hbm = b_hbm = dis8_out = None
        if newton:
            deg_hbm = rest.pop(0)
        elif needs_dis:
            dis_hbm = rest.pop(0)
        if bias:
            b_hbm = rest.pop(0)
        out_hbm = rest.pop(0)
        if newton:
            dis8_out = rest.pop(0)
        src_v, dst_v, rows_v, acc_s, semg, sems, u_s = rest[:7]
        rest = rest[7:]
        dis_v = deg2_v = b_v = None
        if needs_dis:
            dis_v = rest.pop(0)
        if newton:
            deg2_v = rest.pop(0)
        if bias:
            b_v = rest.pop(0)

        c = lax.axis_index("c")
        s = lax.axis_index("s")
        base = s * rows_per_tile
        uc = u_hbm.at[c]
        oc = out_hbm.at[c]

        if newton:
            pltpu.sync_copy(deg_hbm.at[pl.ds(base * 8, rt8)],
                            dis_v.at[pl.ds(0, rt8)])
            pltpu.sync_copy(deg_hbm.at[pl.ds((NPAD + base) * 8, rt8)],
                            deg2_v.at[pl.ds(0, rt8)])

            def nbody(i, carry):
                t = (dis_v[pl.ds(i * 16, 16)] + deg2_v[pl.ds(i * 16, 16)])
                y = jnp.full((16,), 2.0 ** -9.5, jnp.float32)
                for k in range(8, -1, -1):
                    y = jnp.where(t < 4.0 ** (k + 1),
                                  jnp.full((16,), 2.0 ** -(k + 0.5),
                                           jnp.float32), y)
                for _ in range(5):
                    y = y * (1.5 - 0.5 * t * y * y)
                dis_v[pl.ds(i * 16, 16)] = y
                return carry

            lax.fori_loop(0, rt8 // 16, nbody, 0)
            pltpu.sync_copy(dis_v.at[pl.ds(0, rt8)],
                            dis8_out.at[pl.ds(base * 8, rt8)])
        elif needs_dis:
            pltpu.sync_copy(dis_hbm.at[pl.ds(base * 8, rt8)],
                            dis_v.at[pl.ds(0, rt8)])
        if bias:
            pltpu.sync_copy(b_hbm.at[c], b_v)

        def rowscale(k, with_bias):
            rv = rows_v.at[0]

            def rowfix(r, carry):
                dsc = dis_v[pl.ds((k * CHUNK + r) * 8, 16)][0]
                for h in range(F // 16):
                    v = rv[r, pl.ds(h * 16, 16)] * dsc
                    if with_bias:
                        v = v + b_v[pl.ds(h * 16, 16)]
                    rv[r, pl.ds(h * 16, 16)] = v
                return carry

            lax.fori_loop(0, CHUNK, rowfix, 0)

        for k in range(rows_per_tile // CHUNK):
            pltpu.sync_copy(uc.at[pl.ds(base + k * CHUNK, CHUNK)],
                            rows_v.at[0])
            if stage_scale:
                rowscale(k, False)
            pltpu.sync_copy(rows_v.at[0],
                            acc_s.at[pl.ds(base + k * CHUNK, CHUNK)])
            pltpu.sync_copy(rows_v.at[0],
                            u_s.at[pl.ds(base + k * CHUNK, CHUNK)])
        plsc.subcore_barrier()

        def gather(j, slot):
            pltpu.async_copy(u_s.at[src_v.at[j]], rows_v.at[slot],
                             semg.at[slot])

        def gwait(j, slot):
            pltpu.make_async_copy(u_s.at[src_v.at[j]], rows_v.at[slot],
                                  semg.at[slot]).wait()

        def scat(j, slot):
            pltpu.async_copy(rows_v.at[slot], acc_s.at[dst_v.at[j]],
                             sems.at[slot], add=True)

        def swait(j, slot):
            pltpu.make_async_copy(rows_v.at[slot], acc_s.at[dst_v.at[j]],
                                  sems.at[slot]).wait()

        def step(j, b):
            gwait(j, b)
            scat(j, b)
            pb = (b + ahead) % nslots
            if isinstance(j, int):
                if j + ahead - nslots >= 0:
                    swait(j + ahead - nslots, pb)
                if j + ahead < cpp:
                    gather(j + ahead, pb)
            else:
                @pl.when(j + ahead - nslots >= 0)
                def _():
                    swait(j + ahead - nslots, pb)

                @pl.when(j + ahead < cpp)
                def _():
                    gather(j + ahead, pb)

        for phase in range(nphases):
            pltpu.sync_copy(src_hbm.at[s].at[pl.ds(phase * cpp, cpp)], src_v)
            pltpu.sync_copy(dst_hbm.at[s].at[pl.ds(phase * cpp, cpp)], dst_v)
            for b in range(ahead):
                gather(b, b)

            def body(g, carry):
                jn = nslots * g
                for b in range(nslots):
                    step(jn + b, b)
                return carry

            n_grp = cpp // nslots
            lax.fori_loop(0, n_grp, body, 0)
            for j in range(n_grp * nslots, cpp):
                step(j, j % nslots)
            for j in range(cpp - nslots + ahead, cpp):
                swait(j, j % nslots)
        plsc.subcore_barrier()

        for k in range(rows_per_tile // CHUNK):
            pltpu.sync_copy(acc_s.at[pl.ds(base + k * CHUNK, CHUNK)],
                            rows_v.at[0])
            if out_scale:
                rowscale(k, bias)
            pltpu.sync_copy(rows_v.at[0],
                            oc.at[pl.ds(base + k * CHUNK, CHUNK)])

    return prop


_prop64 = _make_prop(64, newton=True, stage_scale=True, out_scale=True,
                     nslots=3, nphases=4)
_prop32 = _make_prop(32, stage_scale=True, out_scale=True, bias=True,
                     nslots=8, nphases=1)


_ROWS = 256


def _mm_body(w_ref, W1a_ref, b1a_ref, W1b_ref, W2a_ref, b2a_ref,
             W2b_ref, o_ref):
    z = jnp.concatenate([w_ref[0], w_ref[1]], axis=1)
    h1 = jnp.maximum(
        jnp.dot(z, W1a_ref[...], preferred_element_type=jnp.float32)
        + b1a_ref[...], 0.0)
    o_ref[0] = jnp.dot(h1, W1b_ref[...], preferred_element_type=jnp.float32)
    h2 = jnp.maximum(
        jnp.dot(z, W2a_ref[...], preferred_element_type=jnp.float32)
        + b2a_ref[...], 0.0)
    o_ref[1] = jnp.dot(h2, W2b_ref[...], preferred_element_type=jnp.float32)


_mm_k = pl.pallas_call(
    _mm_body,
    grid=(NPAD // _ROWS,),
    in_specs=[
        pl.BlockSpec((2, _ROWS, 64), lambda i: (0, i, 0)),
        pl.BlockSpec((128, 256), lambda i: (0, 0)),
        pl.BlockSpec((1, 256), lambda i: (0, 0)),
        pl.BlockSpec((256, 32), lambda i: (0, 0)),
        pl.BlockSpec((128, 256), lambda i: (0, 0)),
        pl.BlockSpec((1, 256), lambda i: (0, 0)),
        pl.BlockSpec((256, 32), lambda i: (0, 0)),
    ],
    out_specs=pl.BlockSpec((2, _ROWS, 32), lambda i: (0, i, 0)),
    out_shape=jax.ShapeDtypeStruct((2, NPAD, 32), jnp.float32),
)


def kernel(x, edge_index, W1a, b1a, W1b, b1b, W2a, b2a, W2b, b2b):
    src = edge_index[0]
    dst = edge_index[1]
    pad = jnp.full((EPAD - E,), N, jnp.int32)
    srcp = jnp.concatenate([src, pad])
    dstp = jnp.concatenate([dst, pad])
    dstA = dstp.reshape(32, 80, CHUNK)
    src16 = srcp.reshape(16, 160, CHUNK)
    dst16 = dstp.reshape(16, 160, CHUNK)
    x_pad = jnp.pad(x, ((0, NPAD - N), (0, 0)))

    degpair = _deg_k(dstA, jnp.full((CHUNK, 8), 1.0, jnp.float32),
                     jnp.full((NPAD, 8), 0.5, jnp.float32))
    xs = jnp.stack([x_pad[:, :64], x_pad[:, 64:]])
    z, dis8 = _prop64(xs, src16, dst16, degpair.reshape(-1))
    u2 = _mm_k(z, W1a, b1a.reshape(1, -1), W1b,
               W2a, b2a.reshape(1, -1), W2b)
    out = _prop32(u2, src16, dst16, dis8, jnp.stack([b1b, b2b]))
    return (out[0, :N], out[1, :N])
